# baseline (speedup 1.0000x reference)
import functools

import jax
import jax.numpy as jnp
from jax import lax
from jax.experimental import pallas as pl
from jax.experimental.pallas import tpu as pltpu
from jax.experimental.pallas import tpu_sc as plsc

_TM = 40
_TN = 256


def _pick_chunk(n: int, cap: int = 10000) -> int:
    for c in range(min(n, cap), 7, -1):
        if n % c == 0 and c % _TM == 0:
            return c
    return n


def _topk_body(n_chunk, n_total, x_ref, w_ref, db_ref, val_ref, idx_ref,
               qn_ref, dn_ref, accv_ref, acci_ref):
    i = pl.program_id(0)
    b, d = x_ref.shape

    @pl.when(i == 0)
    def _init():
        feats = jnp.dot(x_ref[...], w_ref[...],
                        preferred_element_type=jnp.float32)
        qnorm = jnp.sqrt(jnp.sum(feats * feats, axis=1, keepdims=True))
        qn_ref[...] = feats / (qnorm + 1e-8)
        accv_ref[...] = jnp.full(accv_ref.shape, -jnp.inf, jnp.float32)
        acci_ref[...] = jnp.zeros(acci_ref.shape, jnp.int32)

    db = db_ref[...]
    dnorm = jnp.sqrt(jnp.sum(db * db, axis=1, keepdims=True))
    dn_ref[...] = db / (dnorm + 1e-8)

    n_tiles = n_chunk // _TM
    tile_iota = lax.broadcasted_iota(jnp.int32, (_TM, _TN), 0)

    for w in range(b // _TN):
        qn_w = qn_ref[pl.ds(w * _TN, _TN), :]
        ws = slice(w * _TN, (w + 1) * _TN)

        def tile_step(t, carry, qn_w=qn_w):
            av, ai = carry
            dtile = dn_ref[pl.ds(t * _TM, _TM), :]
            st = lax.dot_general(dtile, qn_w, (((1,), (1,)), ((), ())),
                                 preferred_element_type=jnp.float32)
            rid = tile_iota + (i * n_chunk + t * _TM)
            cmp = st > av
            return jnp.maximum(av, st), jnp.where(cmp, rid, ai)

        av, ai = lax.fori_loop(0, n_tiles, tile_step,
                               (accv_ref[:, ws], acci_ref[:, ws]))
        accv_ref[:, ws] = av
        acci_ref[:, ws] = ai

    @pl.when(i == pl.num_programs(0) - 1)
    def _finish():
        v = accv_ref[...]
        ix = acci_ref[...]
        m = jnp.max(v, axis=0)
        cand = jnp.min(jnp.where(v == m[None, :], ix, n_total), axis=0)
        val_ref[...] = m
        idx_ref[...] = cand


def _topk_call(x, W, db):
    b, d = x.shape
    n = db.shape[0]
    chunk = _pick_chunk(n)
    nsteps = n // chunk
    return pl.pallas_call(
        functools.partial(_topk_body, chunk, n),
        grid=(nsteps,),
        in_specs=[
            pl.BlockSpec((b, d), lambda i: (0, 0)),
            pl.BlockSpec((d, d), lambda i: (0, 0)),
            pl.BlockSpec((chunk, d), lambda i: (i, 0)),
        ],
        out_specs=[
            pl.BlockSpec((b,), lambda i: (0,)),
            pl.BlockSpec((b,), lambda i: (0,)),
        ],
        out_shape=[
            jax.ShapeDtypeStruct((b,), jnp.float32),
            jax.ShapeDtypeStruct((b,), jnp.int32),
        ],
        scratch_shapes=[
            pltpu.VMEM((b, d), jnp.float32),
            pltpu.VMEM((chunk, d), jnp.float32),
            pltpu.VMEM((_TM, b), jnp.float32),
            pltpu.VMEM((_TM, b), jnp.int32),
        ],
    )(x, W, db)


def _label_gather(labels, idx):
    b = idx.shape[0]
    info = plsc.get_sparse_core_info()
    nw = info.num_cores * info.num_subcores
    bpw = b // nw
    mesh = plsc.VectorSubcoreMesh(core_axis_name="c", subcore_axis_name="s")

    @functools.partial(
        pl.kernel,
        mesh=mesh,
        out_type=jax.ShapeDtypeStruct((b,), jnp.int32),
        scratch_types=[
            pltpu.VMEM((bpw,), jnp.int32),
            pltpu.VMEM((bpw,), jnp.int32),
            pltpu.SemaphoreType.DMA,
        ],
    )
    def k(labels_hbm, idx_hbm, out_hbm, idx_v, vals_v, sem):
        wid = lax.axis_index("s") * info.num_cores + lax.axis_index("c")
        base = wid * bpw
        pltpu.sync_copy(idx_hbm.at[pl.ds(base, bpw)], idx_v)
        pltpu.async_copy(labels_hbm.at[idx_v], vals_v, sem).wait()
        pltpu.sync_copy(vals_v, out_hbm.at[pl.ds(base, bpw)])

    return k(labels, idx)


def kernel(x, W, db_features, db_labels):
    top_vals, top_idx = _topk_call(x, W, db_features)
    pred = _label_gather(db_labels, top_idx)
    return top_vals, top_idx, pred


# speedup vs baseline: 17.5812x; 17.5812x over previous
import functools

import jax
import jax.numpy as jnp
from jax import lax
from jax.experimental import pallas as pl
from jax.experimental.pallas import tpu as pltpu
from jax.experimental.pallas import tpu_sc as plsc


def _pick_chunk(n: int, cap: int = 5120) -> int:
    for c in range(min(n, cap), 15, -1):
        if n % c == 0 and c % 16 == 0:
            return c
    return n


def _half_topk(i, half, base_row, qn, val_ref, idx_ref):
    dnorm = jnp.sqrt(jnp.sum(half * half, axis=1, keepdims=True))
    dn = half / (dnorm + 1e-8)
    s = lax.dot_general(dn, qn, (((1,), (1,)), ((), ())),
                        preferred_element_type=jnp.float32)
    m = jnp.max(s, axis=0)
    cand = jnp.argmax(s, axis=0).astype(jnp.int32)
    gidx = cand + base_row
    better = m > val_ref[...]
    val_ref[...] = jnp.where(better, m, val_ref[...])
    idx_ref[...] = jnp.where(better, gidx, idx_ref[...])


def _topk_body(n_chunk, x_ref, w_ref, dba_ref, dbb_ref, val_ref, idx_ref,
               qn_ref):
    i = pl.program_id(0)

    @pl.when(i == 0)
    def _init():
        feats = jnp.dot(x_ref[...], w_ref[...],
                        preferred_element_type=jnp.float32)
        qnorm = jnp.sqrt(jnp.sum(feats * feats, axis=1, keepdims=True))
        qn_ref[...] = feats / (qnorm + 1e-8)
        val_ref[...] = jnp.full(val_ref.shape, -jnp.inf, jnp.float32)
        idx_ref[...] = jnp.zeros(idx_ref.shape, jnp.int32)

    half = n_chunk // 2
    qn = qn_ref[...]
    _half_topk(i, dba_ref[...], i * n_chunk, qn, val_ref, idx_ref)
    _half_topk(i, dbb_ref[...], i * n_chunk + half, qn, val_ref, idx_ref)


def _topk_call(x, W, db):
    b, d = x.shape
    n = db.shape[0]
    chunk = _pick_chunk(n)
    half = chunk // 2
    nsteps = n // chunk
    return pl.pallas_call(
        functools.partial(_topk_body, chunk),
        grid=(nsteps,),
        in_specs=[
            pl.BlockSpec((b, d), lambda i: (0, 0)),
            pl.BlockSpec((d, d), lambda i: (0, 0)),
            pl.BlockSpec((half, d), lambda i: (2 * i, 0)),
            pl.BlockSpec((half, d), lambda i: (2 * i + 1, 0)),
        ],
        out_specs=[
            pl.BlockSpec((b,), lambda i: (0,)),
            pl.BlockSpec((b,), lambda i: (0,)),
        ],
        out_shape=[
            jax.ShapeDtypeStruct((b,), jnp.float32),
            jax.ShapeDtypeStruct((b,), jnp.int32),
        ],
        scratch_shapes=[pltpu.VMEM((b, d), jnp.float32)],
    )(x, W, db, db)


def _label_gather(labels, idx):
    b = idx.shape[0]
    info = plsc.get_sparse_core_info()
    nw = info.num_cores * info.num_subcores
    bpw = b // nw
    mesh = plsc.VectorSubcoreMesh(core_axis_name="c", subcore_axis_name="s")

    @functools.partial(
        pl.kernel,
        mesh=mesh,
        out_type=jax.ShapeDtypeStruct((b,), jnp.int32),
        scratch_types=[
            pltpu.VMEM((bpw,), jnp.int32),
            pltpu.VMEM((bpw,), jnp.int32),
            pltpu.SemaphoreType.DMA,
        ],
    )
    def k(labels_hbm, idx_hbm, out_hbm, idx_v, vals_v, sem):
        wid = lax.axis_index("s") * info.num_cores + lax.axis_index("c")
        base = wid * bpw
        pltpu.sync_copy(idx_hbm.at[pl.ds(base, bpw)], idx_v)
        pltpu.async_copy(labels_hbm.at[idx_v], vals_v, sem).wait()
        pltpu.sync_copy(vals_v, out_hbm.at[pl.ds(base, bpw)])

    return k(labels, idx)


def kernel(x, W, db_features, db_labels):
    top_vals, top_idx = _topk_call(x, W, db_features)
    pred = _label_gather(db_labels, top_idx)
    return top_vals, top_idx, pred
